# baseline (speedup 1.0000x reference)
import functools

import jax
import jax.numpy as jnp
from jax import lax
from jax.experimental import pallas as pl
from jax.experimental.pallas import tpu as pltpu
from jax.experimental.pallas import tpu_sc as plsc

D_MODEL = 64
VOCAB = 1000000
VOCAB_PAD = 1000064
BATCH = 4096
SEQ = 200
NUM_WORKERS = 32
BLK = 128
N_TBLK = VOCAB_PAD // BLK
SCALE = 8.0
L = 16
NBUF = 2
PADW = BLK + 1
OPADW = BATCH // NUM_WORKERS + 1

_MESH = plsc.VectorSubcoreMesh(core_axis_name="c", subcore_axis_name="s")
_PARAMS = pltpu.CompilerParams(
    use_tc_tiling_on_sc=True, needs_layout_passes=False)


@functools.partial(
    pl.kernel,
    mesh=_MESH,
    out_type=jax.ShapeDtypeStruct((VOCAB_PAD, BLK), jnp.float32),
    compiler_params=_PARAMS,
    scratch_types=[
        pltpu.VMEM((NBUF, D_MODEL, BLK), jnp.float32),
        pltpu.VMEM((NBUF, BLK, PADW), jnp.float32),
        pltpu.SemaphoreType.DMA((NBUF,)),
        pltpu.SemaphoreType.DMA((NBUF,)),
    ],
)
def _relayout(tt_hbm, t2_hbm, src_v, dst_v, gsem, ssem):
    w = lax.axis_index("s") * 2 + lax.axis_index("c")

    def start_load(t, b):
        pltpu.async_copy(tt_hbm.at[:, pl.ds(t * BLK, BLK)], src_v.at[b],
                         gsem.at[b])

    def wait_load(t, b):
        pltpu.make_async_copy(tt_hbm.at[:, pl.ds(t * BLK, BLK)],
                              src_v.at[b], gsem.at[b]).wait()

    def start_store(t, b):
        pltpu.async_copy(dst_v.at[b, :, pl.ds(0, BLK)],
                         t2_hbm.at[pl.ds(t * BLK, BLK)], ssem.at[b])

    def wait_store(t, b):
        pltpu.make_async_copy(dst_v.at[b, :, pl.ds(0, BLK)],
                              t2_hbm.at[pl.ds(t * BLK, BLK)],
                              ssem.at[b]).wait()

    iota = lax.iota(jnp.int32, L)
    rowid = [(iota + j * L) * PADW for j in range(BLK // L)]

    nblk = (N_TBLK - 1 - w) // NUM_WORKERS + 1

    for b in range(NBUF):
        @pl.when(b < nblk)
        def _():
            start_load(w + b * NUM_WORKERS, b)

    n_rounds = ((nblk + NBUF - 1) // NBUF) * NBUF

    @pl.loop(0, n_rounds, step=NBUF)
    def _outer(i0):
        for b in range(NBUF):
            i = i0 + b

            @pl.when(i < nblk)
            def _():
                t = w + i * NUM_WORKERS
                wait_load(t, b)

                @pl.when(i >= NBUF)
                def _():
                    wait_store(t - NBUF * NUM_WORKERS, b)

                @plsc.parallel_loop(0, D_MODEL, unroll=2)
                def _tp(c):
                    cvec = jnp.full((L,), 0, jnp.int32) + c
                    for j in range(BLK // L):
                        val = src_v[b, c, pl.ds(j * L, L)]
                        plsc.store_scatter(dst_v.at[b], [iota + j * L, cvec],
                                           val * SCALE)

                @pl.when(i + NBUF < nblk)
                def _():
                    start_load(t + NBUF * NUM_WORKERS, b)

                start_store(t, b)

    for b in range(NBUF):
        i = nblk - NBUF + b

        @pl.when(i >= 0)
        def _():
            wait_store(w + i * NUM_WORKERS, b)


@functools.partial(
    pl.kernel,
    mesh=_MESH,
    out_type=jax.ShapeDtypeStruct((SEQ, D_MODEL, BATCH), jnp.float32),
    compiler_params=_PARAMS,
    scratch_types=[
        pltpu.VMEM((SEQ, BLK), jnp.int32),
        pltpu.VMEM((NBUF, BLK, BLK), jnp.float32),
        pltpu.VMEM((NBUF, D_MODEL, OPADW), jnp.float32),
        pltpu.SemaphoreType.DMA((NBUF,)),
        pltpu.SemaphoreType.DMA((NBUF,)),
    ],
)
def _gather(xt_hbm, t2_hbm, out_hbm, idx_v, row_v, out_v, gsem, ssem):
    w = lax.axis_index("s") * 2 + lax.axis_index("c")
    b0 = w * BLK
    pltpu.sync_copy(xt_hbm.at[:, pl.ds(b0, BLK)], idx_v)

    def start_gather(s, b):
        pltpu.async_copy(t2_hbm.at[idx_v.at[s]], row_v.at[b], gsem.at[b])

    def wait_gather(s, b):
        pltpu.make_async_copy(t2_hbm.at[idx_v.at[s]], row_v.at[b],
                              gsem.at[b]).wait()

    def start_store(s, b):
        pltpu.async_copy(out_v.at[b, :, pl.ds(0, BLK)],
                         out_hbm.at[s, :, pl.ds(b0, BLK)], ssem.at[b])

    def wait_store(s, b):
        pltpu.make_async_copy(out_v.at[b, :, pl.ds(0, BLK)],
                              out_hbm.at[s, :, pl.ds(b0, BLK)],
                              ssem.at[b]).wait()

    iota = lax.iota(jnp.int32, L)

    for b in range(NBUF):
        start_gather(b, b)

    @pl.loop(0, SEQ, step=NBUF)
    def _outer(s0):
        for b in range(NBUF):
            s = s0 + b
            wait_gather(s, b)

            @pl.when(s >= NBUF)
            def _():
                wait_store(s - NBUF, b)

            @plsc.parallel_loop(0, BLK, unroll=4)
            def _shuffle(bb):
                bvec = jnp.full((L,), 0, jnp.int32) + bb
                for j in range(D_MODEL // L):
                    val = row_v[b, bb, pl.ds(j * L, L)]
                    plsc.store_scatter(out_v.at[b], [iota + j * L, bvec], val)

            @pl.when(s + NBUF < SEQ)
            def _():
                start_gather(s + NBUF, b)

            start_store(s, b)

    for b in range(NBUF):
        wait_store(SEQ - NBUF + b, b)


def kernel(x, table):
    t2 = _relayout(table.T)
    xt = x.astype(jnp.int32).T
    out3 = _gather(xt, t2)
    return jnp.transpose(out3, (2, 0, 1))


# speedup vs baseline: 1.0807x; 1.0807x over previous
import functools

import jax
import jax.numpy as jnp
from jax import lax
from jax.experimental import pallas as pl
from jax.experimental.pallas import tpu as pltpu
from jax.experimental.pallas import tpu_sc as plsc

D_MODEL = 64
VOCAB_PAD = 1000064
BATCH = 4096
SEQ = 200
NUM_WORKERS = 32
BLK = 128
N_TBLK = VOCAB_PAD // BLK
SCALE = 8.0
L = 16
NBUF = 2
PADW = BLK + 1

_MESH = plsc.VectorSubcoreMesh(core_axis_name="c", subcore_axis_name="s")
_PARAMS = pltpu.CompilerParams(
    use_tc_tiling_on_sc=True, needs_layout_passes=False)


@functools.partial(
    pl.kernel,
    mesh=_MESH,
    out_type=jax.ShapeDtypeStruct((VOCAB_PAD, BLK), jnp.float32),
    compiler_params=_PARAMS,
    scratch_types=[
        pltpu.VMEM((NBUF, D_MODEL, PADW), jnp.float32),
        pltpu.VMEM((NBUF, BLK, BLK), jnp.float32),
        pltpu.SemaphoreType.DMA((NBUF,)),
        pltpu.SemaphoreType.DMA((NBUF,)),
    ],
)
def _relayout(tt_hbm, t2_hbm, src_v, dst_v, gsem, ssem):
    w = lax.axis_index("s") * 2 + lax.axis_index("c")

    def start_load(t, b):
        pltpu.async_copy(tt_hbm.at[:, pl.ds(t * BLK, BLK)],
                         src_v.at[b, :, pl.ds(0, BLK)], gsem.at[b])

    def wait_load(t, b):
        pltpu.make_async_copy(tt_hbm.at[:, pl.ds(t * BLK, BLK)],
                              src_v.at[b, :, pl.ds(0, BLK)], gsem.at[b]).wait()

    def start_store(t, b):
        pltpu.async_copy(dst_v.at[b], t2_hbm.at[pl.ds(t * BLK, BLK)],
                         ssem.at[b])

    def wait_store(t, b):
        pltpu.make_async_copy(dst_v.at[b], t2_hbm.at[pl.ds(t * BLK, BLK)],
                              ssem.at[b]).wait()

    iota = lax.iota(jnp.int32, L)
    crow = [iota + l * L for l in range(D_MODEL // L)]

    nblk = (N_TBLK - 1 - w) // NUM_WORKERS + 1
    n_rounds = ((nblk + NBUF - 1) // NBUF) * NBUF

    for b in range(NBUF):
        @pl.when(b < nblk)
        def _():
            start_load(w + b * NUM_WORKERS, b)

    @pl.loop(0, n_rounds, step=NBUF)
    def _outer(i0):
        for b in range(NBUF):
            i = i0 + b

            @pl.when(i < nblk)
            def _():
                t = w + i * NUM_WORKERS
                wait_load(t, b)

                @pl.when(i >= NBUF)
                def _():
                    wait_store(t - NBUF * NUM_WORKERS, b)

                @plsc.parallel_loop(0, BLK, unroll=4)
                def _tp(p):
                    pvec = jnp.full((L,), 0, jnp.int32) + p
                    for l in range(D_MODEL // L):
                        val = plsc.load_gather(src_v.at[b], [crow[l], pvec])
                        dst_v[b, p, pl.ds(l * L, L)] = val * SCALE

                @pl.when(i + NBUF < nblk)
                def _():
                    start_load(t + NBUF * NUM_WORKERS, b)

                start_store(t, b)

    for b in range(NBUF):
        i = nblk - NBUF + b

        @pl.when(i >= 0)
        def _():
            wait_store(w + i * NUM_WORKERS, b)


@functools.partial(
    pl.kernel,
    mesh=_MESH,
    out_type=jax.ShapeDtypeStruct((SEQ, D_MODEL, BATCH), jnp.float32),
    compiler_params=_PARAMS,
    scratch_types=[
        pltpu.VMEM((SEQ, BLK), jnp.int32),
        pltpu.VMEM((NBUF, BLK, PADW), jnp.float32),
        pltpu.VMEM((NBUF, D_MODEL, BLK), jnp.float32),
        pltpu.SemaphoreType.DMA((NBUF,)),
        pltpu.SemaphoreType.DMA((NBUF,)),
    ],
)
def _gather(xt_hbm, t2_hbm, out_hbm, idx_v, row_v, out_v, gsem, ssem):
    w = lax.axis_index("s") * 2 + lax.axis_index("c")
    b0 = w * BLK
    pltpu.sync_copy(xt_hbm.at[:, pl.ds(b0, BLK)], idx_v)

    def start_gather(s, b):
        pltpu.async_copy(t2_hbm.at[idx_v.at[s]],
                         row_v.at[b, :, pl.ds(0, BLK)], gsem.at[b])

    def wait_gather(s, b):
        pltpu.make_async_copy(t2_hbm.at[idx_v.at[s]],
                              row_v.at[b, :, pl.ds(0, BLK)], gsem.at[b]).wait()

    def start_store(s, b):
        pltpu.async_copy(out_v.at[b], out_hbm.at[s, :, pl.ds(b0, BLK)],
                         ssem.at[b])

    def wait_store(s, b):
        pltpu.make_async_copy(out_v.at[b], out_hbm.at[s, :, pl.ds(b0, BLK)],
                              ssem.at[b]).wait()

    iota = lax.iota(jnp.int32, L)
    brow = [iota + l * L for l in range(BLK // L)]

    for b in range(NBUF):
        start_gather(b, b)

    @pl.loop(0, SEQ, step=NBUF)
    def _outer(s0):
        for b in range(NBUF):
            s = s0 + b
            wait_gather(s, b)

            @pl.when(s >= NBUF)
            def _():
                wait_store(s - NBUF, b)

            @plsc.parallel_loop(0, D_MODEL, unroll=4)
            def _shuffle(c):
                cvec = jnp.full((L,), 0, jnp.int32) + c
                for l in range(BLK // L):
                    val = plsc.load_gather(row_v.at[b], [brow[l], cvec])
                    out_v[b, c, pl.ds(l * L, L)] = val

            @pl.when(s + NBUF < SEQ)
            def _():
                start_gather(s + NBUF, b)

            start_store(s, b)

    for b in range(NBUF):
        wait_store(SEQ - NBUF + b, b)


def kernel(x, table):
    t2 = _relayout(table.T)
    xt = x.astype(jnp.int32).T
    out3 = _gather(xt, t2)
    return jnp.transpose(out3, (2, 0, 1))


# speedup vs baseline: 1.2887x; 1.1925x over previous
import functools

import jax
import jax.numpy as jnp
from jax import lax
from jax.experimental import pallas as pl
from jax.experimental.pallas import tpu as pltpu
from jax.experimental.pallas import tpu_sc as plsc

D_MODEL = 64
B_TOTAL = 4096 * 200
NUM_WORKERS = 32
B_PER_W = B_TOTAL // NUM_WORKERS
CHUNK = 128
N_CHUNKS = B_PER_W // CHUNK
SCALE = 8.0
LANES = 16
NBUF = 4


@functools.partial(
    pl.kernel,
    mesh=plsc.VectorSubcoreMesh(core_axis_name="c", subcore_axis_name="s"),
    out_type=jax.ShapeDtypeStruct((B_TOTAL, D_MODEL), jnp.float32),
    compiler_params=pltpu.CompilerParams(use_tc_tiling_on_sc=False),
    scratch_types=[
        pltpu.VMEM((N_CHUNKS, CHUNK), jnp.int32),
        pltpu.VMEM((NBUF, CHUNK, D_MODEL), jnp.float32),
        pltpu.VMEM((NBUF, CHUNK, D_MODEL), jnp.float32),
        pltpu.SemaphoreType.DMA((NBUF,)),
        pltpu.SemaphoreType.DMA((NBUF,)),
    ],
)
def _emb_lookup(idx_hbm, table_hbm, out_hbm, idx_v, buf_g, buf_s, gsem, ssem):
    wid = lax.axis_index("s") * 2 + lax.axis_index("c")
    base = wid * B_PER_W
    pltpu.sync_copy(idx_hbm.at[pl.ds(wid * N_CHUNKS, N_CHUNKS)], idx_v)

    def start_gather(g, b):
        pltpu.async_copy(table_hbm.at[idx_v.at[g]], buf_g.at[b], gsem.at[b])

    for b in range(NBUF):
        start_gather(b, b)

    @pl.loop(0, N_CHUNKS, step=NBUF)
    def _outer(i0):
        for b in range(NBUF):
            i = i0 + b
            pltpu.make_async_copy(table_hbm.at[idx_v.at[i]],
                                  buf_g.at[b], gsem.at[b]).wait()
            @pl.when(i >= NBUF)
            def _():
                pltpu.make_async_copy(
                    buf_s.at[b],
                    out_hbm.at[pl.ds(base + (i - NBUF) * CHUNK, CHUNK)],
                    ssem.at[b]).wait()

            def row_body(r, c):
                for j in range(D_MODEL // LANES):
                    sl = pl.ds(j * LANES, LANES)
                    buf_s[b, r, sl] = buf_g[b, r, sl] * SCALE
                return c

            lax.fori_loop(0, CHUNK, row_body, 0)

            @pl.when(i + NBUF < N_CHUNKS)
            def _():
                start_gather(i + NBUF, b)

            pltpu.async_copy(
                buf_s.at[b],
                out_hbm.at[pl.ds(base + i * CHUNK, CHUNK)],
                ssem.at[b])

    for b in range(NBUF):
        i = N_CHUNKS - NBUF + b
        pltpu.make_async_copy(
            buf_s.at[b],
            out_hbm.at[pl.ds(base + i * CHUNK, CHUNK)],
            ssem.at[b]).wait()


def kernel(x, table):
    idx = x.reshape(-1).astype(jnp.int32).reshape(-1, CHUNK)
    out = _emb_lookup(idx, table)
    return out.reshape(x.shape + (D_MODEL,))
